# baseline (speedup 1.0000x reference)
import functools

import jax
import jax.numpy as jnp
from jax import lax
from jax.experimental import pallas as pl
from jax.experimental.pallas import tpu as pltpu
from jax.experimental.pallas import tpu_sc as plsc

EPS = 1e-5

_NW = 32
_KROWS = 13
_TROWS = 136


def _sc_counts_body(batch_hbm, out_hbm, idx_v, tbl_v):
    c = lax.axis_index("c")
    s = lax.axis_index("s")
    wid = s * 2 + c
    pltpu.sync_copy(batch_hbm.at[wid], idx_v)

    def zero_row(i, _):
        tbl_v[pl.ds(i * 16, 16)] = jnp.zeros((16,), jnp.float32)
        return 0

    lax.fori_loop(0, _TROWS, zero_row, 0)

    lanes = lax.broadcasted_iota(jnp.int32, (16,), 0)
    ones = jnp.ones((16,), jnp.float32)

    def add_chunk(i, _):
        j = i // 8
        k = i - 8 * j
        idx16 = idx_v[j, pl.ds(k * 16, 16)]
        plsc.addupdate_scatter(tbl_v, [idx16 * 16 + lanes], ones)
        return 0

    lax.fori_loop(0, _KROWS * 8, add_chunk, 0)
    pltpu.sync_copy(tbl_v, out_hbm.at[wid])


def _sc_counts(batch3):
    mesh = plsc.VectorSubcoreMesh(core_axis_name="c", subcore_axis_name="s",
                                  num_cores=2, num_subcores=16)
    return pl.kernel(
        _sc_counts_body,
        out_type=jax.ShapeDtypeStruct((_NW, _TROWS * 16), jnp.float32),
        mesh=mesh,
        scratch_types=[
            pltpu.VMEM((_KROWS, 128), jnp.int32),
            pltpu.VMEM((_TROWS * 16,), jnp.float32),
        ],
        compiler_params=pltpu.CompilerParams(needs_layout_passes=False),
    )(batch3)


def _main_body(nb, bsz, r, batch_ref, x_ref, vn0_ref, xout_ref, acc_ref):
    i = pl.program_id(0)

    @pl.when(i == 0)
    def _init():
        acc_ref[...] = jnp.zeros_like(acc_ref)

    vn0 = vn0_ref[0, :]
    xb = x_ref[...]
    xout_ref[...] = xb + vn0[None, :]

    seg = batch_ref[0, 0, :]
    hot = (jax.lax.broadcasted_iota(jnp.int32, (r, bsz), 1)
           == seg[:, None]).astype(jnp.float32)
    acc_ref[...] += jax.lax.dot_general(
        hot, xb, (((0,), (0,)), ((), ())),
        preferred_element_type=jnp.float32)


def _epilogue_body(bsz, cnt_ref, acc_ref, vn0_ref, W1_ref, b1_ref, g1_ref,
                   be1_ref, W2_ref, b2_ref, g2_ref, be2_ref, vnout_ref):
    vn0 = vn0_ref[0, :]
    counts = jnp.sum(cnt_ref[...], axis=(0, 2))[:bsz]
    vn_agg = (acc_ref[...] / jnp.clip(counts, 1.0)[:, None]
              + vn0[None, :])

    def bn_relu(h, gamma, beta):
        mu = jnp.mean(h, axis=0)
        var = jnp.mean((h - mu) * (h - mu), axis=0)
        hn = (h - mu) / jnp.sqrt(var + EPS) * gamma[None, :] + beta[None, :]
        return jnp.maximum(hn, 0.0)

    h = jax.lax.dot_general(vn_agg, W1_ref[...], (((1,), (1,)), ((), ())),
                            preferred_element_type=jnp.float32)
    h = bn_relu(h + b1_ref[0, :][None, :], g1_ref[0, :], be1_ref[0, :])
    h = jax.lax.dot_general(h, W2_ref[...], (((1,), (1,)), ((), ())),
                            preferred_element_type=jnp.float32)
    h = bn_relu(h + b2_ref[0, :][None, :], g2_ref[0, :], be2_ref[0, :])
    vnout_ref[...] = vn0[None, :] + h


def kernel(x, batch, vn_weight, W1, b1, gamma1, beta1, W2, b2, gamma2, beta2):
    n, d = x.shape
    bsz = 128
    r = 5000
    assert n % r == 0
    nb = n // r

    batch_i = batch.astype(jnp.int32)
    cnt_partials = jnp.zeros((_NW, _TROWS, 16), jnp.float32).at[0, :bsz, 0].set(
        jax.ops.segment_sum(jnp.ones((n,), jnp.float32), batch_i,
                            num_segments=bsz))

    batch3 = batch_i.reshape(nb, 1, r)
    row = lambda v: v.reshape(1, d)
    full = lambda shape: pl.BlockSpec(shape, lambda i: (0,) * len(shape))

    x_out, sums = pl.pallas_call(
        functools.partial(_main_body, nb, bsz, r),
        grid=(nb,),
        in_specs=[
            pl.BlockSpec((1, 1, r), lambda i: (i, 0, 0)),
            pl.BlockSpec((r, d), lambda i: (i, 0)),
            full((1, d)),
        ],
        out_specs=[
            pl.BlockSpec((r, d), lambda i: (i, 0)),
            pl.BlockSpec((bsz, d), lambda i: (0, 0)),
        ],
        out_shape=[
            jax.ShapeDtypeStruct((n, d), jnp.float32),
            jax.ShapeDtypeStruct((bsz, d), jnp.float32),
        ],
        compiler_params=pltpu.CompilerParams(
            dimension_semantics=("arbitrary",),
        ),
    )(batch3, x, vn_weight)

    full0 = lambda shape: pl.BlockSpec(shape, lambda: (0,) * len(shape))
    vn_out = pl.pallas_call(
        functools.partial(_epilogue_body, bsz),
        in_specs=[
            full0(cnt_partials.shape),
            full0((bsz, d)), full0((1, d)),
            full0((d, d)), full0((1, d)), full0((1, d)), full0((1, d)),
            full0((d, d)), full0((1, d)), full0((1, d)), full0((1, d)),
        ],
        out_specs=pl.BlockSpec((bsz, d), lambda: (0, 0)),
        out_shape=jax.ShapeDtypeStruct((bsz, d), jnp.float32),
    )(cnt_partials, sums, vn_weight, W1, row(b1), row(gamma1), row(beta1),
      W2, row(b2), row(gamma2), row(beta2))
    return (x_out, vn_out)


# speedup vs baseline: 2.0311x; 2.0311x over previous
import functools

import jax
import jax.numpy as jnp
from jax.experimental import pallas as pl
from jax.experimental.pallas import tpu as pltpu

EPS = 1e-5


def _fused_body(nb, bsz, r, batch_ref, x_ref, vn0_ref, W1_ref, b1_ref,
                g1_ref, be1_ref, W2_ref, b2_ref, g2_ref, be2_ref,
                xout_ref, vnout_ref, acc_ref, cnt_ref):
    i = pl.program_id(0)

    @pl.when(i == 0)
    def _init():
        acc_ref[...] = jnp.zeros_like(acc_ref)
        cnt_ref[...] = jnp.zeros_like(cnt_ref)

    vn0 = vn0_ref[0, :]
    xout_ref[...] = x_ref[...] + vn0[None, :]

    c = r // 4
    acc = acc_ref[...]
    cnt = cnt_ref[0, :]
    for k in range(4):
        seg = batch_ref[0, k, :]
        onehot = (jax.lax.broadcasted_iota(jnp.int32, (c, bsz), 1)
                  == seg[:, None]).astype(jnp.float32)
        acc += jax.lax.dot_general(
            onehot, x_ref[k * c:(k + 1) * c, :], (((0,), (0,)), ((), ())),
            preferred_element_type=jnp.float32)
        cnt += jnp.sum(onehot, axis=0)
    acc_ref[...] = acc
    cnt_ref[0, :] = cnt

    @pl.when(i == nb - 1)
    def _epilogue():
        counts = cnt_ref[0, :]
        vn_agg = (acc_ref[...] / jnp.clip(counts, 1.0)[:, None]
                  + vn0[None, :])

        def bn_relu(h, gamma, beta):
            mu = jnp.mean(h, axis=0)
            var = jnp.mean((h - mu) * (h - mu), axis=0)
            hn = (h - mu) / jnp.sqrt(var + EPS) * gamma[None, :] + beta[None, :]
            return jnp.maximum(hn, 0.0)

        h = jax.lax.dot_general(vn_agg, W1_ref[...], (((1,), (1,)), ((), ())),
                                preferred_element_type=jnp.float32)
        h = bn_relu(h + b1_ref[0, :][None, :], g1_ref[0, :], be1_ref[0, :])
        h = jax.lax.dot_general(h, W2_ref[...], (((1,), (1,)), ((), ())),
                                preferred_element_type=jnp.float32)
        h = bn_relu(h + b2_ref[0, :][None, :], g2_ref[0, :], be2_ref[0, :])
        vnout_ref[...] = vn0[None, :] + h


def kernel(x, batch, vn_weight, W1, b1, gamma1, beta1, W2, b2, gamma2, beta2):
    n, d = x.shape
    bsz = 128
    r = 5000
    assert n % r == 0
    nb = n // r

    batch3 = batch.astype(jnp.int32).reshape(nb, 4, r // 4)
    row = lambda v: v.reshape(1, d)

    full = lambda shape: pl.BlockSpec(shape, lambda i: (0,) * len(shape))
    grid_spec = pltpu.PrefetchScalarGridSpec(
        num_scalar_prefetch=0,
        grid=(nb,),
        in_specs=[
            pl.BlockSpec((1, 4, r // 4), lambda i: (i, 0, 0)),
            pl.BlockSpec((r, d), lambda i: (i, 0)),
            full((1, d)),
            full((d, d)), full((1, d)), full((1, d)), full((1, d)),
            full((d, d)), full((1, d)), full((1, d)), full((1, d)),
        ],
        out_specs=[
            pl.BlockSpec((r, d), lambda i: (i, 0)),
            pl.BlockSpec((bsz, d), lambda i: (0, 0)),
        ],
        scratch_shapes=[
            pltpu.VMEM((bsz, d), jnp.float32),
            pltpu.VMEM((1, bsz), jnp.float32),
        ],
    )

    x_out, vn_out = pl.pallas_call(
        functools.partial(_fused_body, nb, bsz, r),
        grid_spec=grid_spec,
        out_shape=[
            jax.ShapeDtypeStruct((n, d), jnp.float32),
            jax.ShapeDtypeStruct((bsz, d), jnp.float32),
        ],
        compiler_params=pltpu.CompilerParams(
            dimension_semantics=("arbitrary",),
        ),
    )(batch3, x, vn_weight, W1, row(b1), row(gamma1), row(beta1),
      W2, row(b2), row(gamma2), row(beta2))
    return (x_out, vn_out)


# speedup vs baseline: 2.0508x; 1.0097x over previous
import functools

import jax
import jax.numpy as jnp
from jax.experimental import pallas as pl
from jax.experimental.pallas import tpu as pltpu

EPS = 1e-5


def _fused_body(nb, bsz, r, batch_ref, x_ref, vn0_ref, W1_ref, b1_ref,
                g1_ref, be1_ref, W2_ref, b2_ref, g2_ref, be2_ref,
                xout_ref, vnout_ref, acc_ref, cnt_ref):
    i = pl.program_id(0)

    @pl.when(i == 0)
    def _init():
        acc_ref[...] = jnp.zeros_like(acc_ref)
        cnt_ref[...] = jnp.zeros_like(cnt_ref)

    vn0 = vn0_ref[0, :]
    xo = x_ref[...] + vn0[None, :]
    xout_ref[...] = xo

    seg = batch_ref[0, 0, :]
    onehot = (jax.lax.broadcasted_iota(jnp.int32, (r, bsz), 1)
              == seg[:, None]).astype(jnp.float32)
    acc_ref[...] += jax.lax.dot_general(
        onehot, xo, (((0,), (0,)), ((), ())),
        preferred_element_type=jnp.float32)
    cnt_ref[0, :] += jnp.sum(onehot, axis=0)

    @pl.when(i == nb - 1)
    def _epilogue():
        counts = cnt_ref[0, :]
        vn_agg = acc_ref[...] / jnp.clip(counts, 1.0)[:, None]

        def bn_relu(h, gamma, beta):
            mu = jnp.mean(h, axis=0)
            var = jnp.mean((h - mu) * (h - mu), axis=0)
            hn = (h - mu) / jnp.sqrt(var + EPS) * gamma[None, :] + beta[None, :]
            return jnp.maximum(hn, 0.0)

        h = jax.lax.dot_general(vn_agg, W1_ref[...], (((1,), (1,)), ((), ())),
                                preferred_element_type=jnp.float32)
        h = bn_relu(h + b1_ref[0, :][None, :], g1_ref[0, :], be1_ref[0, :])
        h = jax.lax.dot_general(h, W2_ref[...], (((1,), (1,)), ((), ())),
                                preferred_element_type=jnp.float32)
        h = bn_relu(h + b2_ref[0, :][None, :], g2_ref[0, :], be2_ref[0, :])
        vnout_ref[...] = vn0[None, :] + h


def kernel(x, batch, vn_weight, W1, b1, gamma1, beta1, W2, b2, gamma2, beta2):
    n, d = x.shape
    bsz = 128
    r = 5000
    assert n % r == 0
    nb = n // r

    batch3 = batch.astype(jnp.int32).reshape(nb, 1, r)
    row = lambda v: v.reshape(1, d)

    full = lambda shape: pl.BlockSpec(shape, lambda i: (0,) * len(shape))
    grid_spec = pltpu.PrefetchScalarGridSpec(
        num_scalar_prefetch=0,
        grid=(nb,),
        in_specs=[
            pl.BlockSpec((1, 1, r), lambda i: (i, 0, 0)),
            pl.BlockSpec((r, d), lambda i: (i, 0)),
            full((1, d)),
            full((d, d)), full((1, d)), full((1, d)), full((1, d)),
            full((d, d)), full((1, d)), full((1, d)), full((1, d)),
        ],
        out_specs=[
            pl.BlockSpec((r, d), lambda i: (i, 0)),
            pl.BlockSpec((bsz, d), lambda i: (0, 0)),
        ],
        scratch_shapes=[
            pltpu.VMEM((bsz, d), jnp.float32),
            pltpu.VMEM((1, bsz), jnp.float32),
        ],
    )

    x_out, vn_out = pl.pallas_call(
        functools.partial(_fused_body, nb, bsz, r),
        grid_spec=grid_spec,
        out_shape=[
            jax.ShapeDtypeStruct((n, d), jnp.float32),
            jax.ShapeDtypeStruct((bsz, d), jnp.float32),
        ],
        compiler_params=pltpu.CompilerParams(
            dimension_semantics=("arbitrary",),
        ),
    )(batch3, x, vn_weight, W1, row(b1), row(gamma1), row(beta1),
      W2, row(b2), row(gamma2), row(beta2))
    return (x_out, vn_out)
